# baseline (speedup 1.0000x reference)
import functools

import jax
import jax.numpy as jnp
from jax import lax
from jax.experimental import pallas as pl
from jax.experimental.pallas import tpu as pltpu
from jax.experimental.pallas import tpu_sc as plsc

N = 10000
E = 320000
H = 128
OUT = 64

NC = 2
NS = 16
NW = NC * NS
EDGES_PER_W = E // NW
CHUNK = 125
NCHUNK = EDGES_PER_W // CHUNK
ROWS_A = 624
ROWS_TAIL = N - NS * ROWS_A

_mesh = plsc.VectorSubcoreMesh(core_axis_name="c", subcore_axis_name="s")


HALF = NCHUNK // 2
NPAIR = HALF // 2


def _zero_rows(src_hbm, dst_sh, s):
    r0 = s * ROWS_A
    tail0 = NS * ROWS_A
    pltpu.sync_copy(src_hbm.at[pl.ds(r0, ROWS_A)], dst_sh.at[pl.ds(r0, ROWS_A)])

    @pl.when(s == NS - 1)
    def _():
        pltpu.sync_copy(src_hbm.at[pl.ds(tail0, ROWS_TAIL)],
                        dst_sh.at[pl.ds(tail0, ROWS_TAIL)])


def _writeback_rows(src_sh, dst_hbm, c, s):
    r0 = s * ROWS_A
    tail0 = NS * ROWS_A
    pltpu.sync_copy(src_sh.at[pl.ds(r0, ROWS_A)],
                    dst_hbm.at[c, pl.ds(r0, ROWS_A)])

    @pl.when(s == NS - 1)
    def _():
        pltpu.sync_copy(src_sh.at[pl.ds(tail0, ROWS_TAIL)],
                        dst_hbm.at[c, pl.ds(tail0, ROWS_TAIL)])


def _segsum_body(with_deg, *refs):
    if with_deg:
        (a_hbm, srcr, dstr, zrow, ones, out_hbm, deg_hbm,
         src_v, dst_v, rows0, rows1, agg_sh, sem0, sem1) = refs
    else:
        (a_hbm, srcr, dstr, zrow, out_hbm,
         src_v, dst_v, rows0, rows1, agg_sh, sem0, sem1) = refs

    c = lax.axis_index("c")
    s = lax.axis_index("s")
    wid = s * NC + c

    _zero_rows(zrow, agg_sh, s)
    plsc.subcore_barrier()

    for half in range(2):
        h0 = half * HALF
        pltpu.sync_copy(srcr.at[wid, pl.ds(h0, HALF)], src_v)
        pltpu.sync_copy(dstr.at[wid, pl.ds(h0, HALF)], dst_v)
        pltpu.async_copy(a_hbm.at[src_v.at[0]], rows0, sem0)

        def pair(p, carry):
            j = p * 2
            pltpu.async_copy(a_hbm.at[src_v.at[j + 1]], rows1, sem1)
            pltpu.make_async_copy(a_hbm.at[src_v.at[j]], rows0, sem0).wait()
            pltpu.sync_copy(rows0, agg_sh.at[dst_v.at[j]], add=True)

            @pl.when(p < NPAIR - 1)
            def _():
                pltpu.async_copy(a_hbm.at[src_v.at[j + 2]], rows0, sem0)

            pltpu.make_async_copy(a_hbm.at[src_v.at[j + 1]], rows1, sem1).wait()
            pltpu.sync_copy(rows1, agg_sh.at[dst_v.at[j + 1]], add=True)
            return carry
        lax.fori_loop(0, NPAIR, pair, 0)

    plsc.subcore_barrier()
    _writeback_rows(agg_sh, out_hbm, c, s)

    if with_deg:
        plsc.subcore_barrier()
        _zero_rows(zrow, agg_sh, s)
        pltpu.sync_copy(ones, rows0)
        plsc.subcore_barrier()

        for half in range(2):
            h0 = half * HALF
            pltpu.sync_copy(dstr.at[wid, pl.ds(h0, HALF)], dst_v)

            def dfire(j, carry):
                pltpu.async_copy(rows0, agg_sh.at[dst_v.at[j]], sem0, add=True)
                return carry
            lax.fori_loop(0, HALF, dfire, 0)

            def ddrain(j, carry):
                pltpu.make_async_copy(rows0, agg_sh.at[dst_v.at[j]],
                                      sem0).wait()
                return carry
            lax.fori_loop(0, HALF, ddrain, 0)

        plsc.subcore_barrier()
        _writeback_rows(agg_sh, deg_hbm, c, s)


_seg_scratch = [
    pltpu.VMEM((HALF, CHUNK), jnp.int32),
    pltpu.VMEM((HALF, CHUNK), jnp.int32),
    pltpu.VMEM((CHUNK, H), jnp.float32),
    pltpu.VMEM((CHUNK, H), jnp.float32),
    pltpu.VMEM_SHARED((N, H), jnp.float32),
    pltpu.SemaphoreType.DMA,
    pltpu.SemaphoreType.DMA,
]

_segsum_deg = pl.kernel(
    functools.partial(_segsum_body, True),
    out_type=(jax.ShapeDtypeStruct((NC, N, H), jnp.float32),
              jax.ShapeDtypeStruct((NC, N, H), jnp.float32)),
    mesh=_mesh,
    scratch_types=_seg_scratch,
)

_segsum = pl.kernel(
    functools.partial(_segsum_body, False),
    out_type=jax.ShapeDtypeStruct((NC, N, H), jnp.float32),
    mesh=_mesh,
    scratch_types=_seg_scratch,
)


BLK = 1000


def _dotT(x, w):
    return lax.dot_general(x, w, (((1,), (1,)), ((), ())),
                           preferred_element_type=jnp.float32)


def _combine2_body(sa_ref, sb_ref, da_ref, db_ref, x_ref, wl_ref, wr_ref,
                   bl_ref, h_ref):
    deg = jnp.maximum(da_ref[...][:, 0:1] + db_ref[...][:, 0:1], 1.0)
    mean = (sa_ref[...] + sb_ref[...]) / deg
    h_ref[...] = jnp.maximum(
        _dotT(mean, wl_ref[...]) + bl_ref[...] + _dotT(x_ref[...], wr_ref[...]),
        0.0)


def _combine2(sa, sb, da, db, x, wl, wr, bl):
    return pl.pallas_call(
        _combine2_body,
        grid=(N // BLK,),
        in_specs=[
            pl.BlockSpec((BLK, H), lambda i: (i, 0)),
            pl.BlockSpec((BLK, H), lambda i: (i, 0)),
            pl.BlockSpec((BLK, H), lambda i: (i, 0)),
            pl.BlockSpec((BLK, H), lambda i: (i, 0)),
            pl.BlockSpec((BLK, H), lambda i: (i, 0)),
            pl.BlockSpec((H, H), lambda i: (0, 0)),
            pl.BlockSpec((H, H), lambda i: (0, 0)),
            pl.BlockSpec((1, H), lambda i: (0, 0)),
        ],
        out_specs=pl.BlockSpec((BLK, H), lambda i: (i, 0)),
        out_shape=jax.ShapeDtypeStruct((N, H), jnp.float32),
    )(sa, sb, da, db, x, wl, wr, bl.reshape(1, H))


def _final_body(sa_ref, sb_ref, da_ref, db_ref, h_ref, wl_ref, wr_ref,
                bl_ref, wc_ref, bc_ref, o_ref):
    deg = jnp.maximum(da_ref[...][:, 0:1] + db_ref[...][:, 0:1], 1.0)
    mean = (sa_ref[...] + sb_ref[...]) / deg
    z = _dotT(mean, wl_ref[...]) + bl_ref[...] + _dotT(h_ref[...], wr_ref[...])
    o_ref[...] = _dotT(z, wc_ref[...]) + bc_ref[...]


def _final(sa, sb, da, db, h, wl, wr, bl, wc, bc):
    return pl.pallas_call(
        _final_body,
        grid=(N // BLK,),
        in_specs=[
            pl.BlockSpec((BLK, H), lambda i: (i, 0)),
            pl.BlockSpec((BLK, H), lambda i: (i, 0)),
            pl.BlockSpec((BLK, H), lambda i: (i, 0)),
            pl.BlockSpec((BLK, H), lambda i: (i, 0)),
            pl.BlockSpec((BLK, H), lambda i: (i, 0)),
            pl.BlockSpec((H, H), lambda i: (0, 0)),
            pl.BlockSpec((H, H), lambda i: (0, 0)),
            pl.BlockSpec((1, H), lambda i: (0, 0)),
            pl.BlockSpec((OUT, H), lambda i: (0, 0)),
            pl.BlockSpec((1, OUT), lambda i: (0, 0)),
        ],
        out_specs=pl.BlockSpec((BLK, OUT), lambda i: (i, 0)),
        out_shape=jax.ShapeDtypeStruct((N, OUT), jnp.float32),
    )(sa, sb, da, db, h, wl, wr, bl.reshape(1, H), wc, bc.reshape(1, OUT))


def kernel(x, edge_index, W1l, b1, W1r, W2l, b2, W2r, Wc, bc):
    srcr = edge_index[0].reshape(NW, NCHUNK, CHUNK)
    dstr = edge_index[1].reshape(NW, NCHUNK, CHUNK)
    zrow = jnp.zeros((N, H), jnp.float32)
    ones = jnp.ones((CHUNK, H), jnp.float32)

    s1p, degp = _segsum_deg(x, srcr, dstr, zrow, ones)
    h = _combine2(s1p[0], s1p[1], degp[0], degp[1], x, W1l, W1r, b1)
    s2p = _segsum(h, srcr, dstr, zrow)
    return _final(s2p[0], s2p[1], degp[0], degp[1], h, W2l, W2r, b2, Wc, bc)


# speedup vs baseline: 1.0096x; 1.0096x over previous
import functools

import jax
import jax.numpy as jnp
from jax import lax
from jax.experimental import pallas as pl
from jax.experimental.pallas import tpu as pltpu
from jax.experimental.pallas import tpu_sc as plsc

N = 10000
E = 320000
H = 128
OUT = 64

NC = 2
NS = 16
NW = NC * NS
EDGES_PER_W = E // NW
CHUNK = 125
NCHUNK = EDGES_PER_W // CHUNK
ROWS_A = 624
ROWS_TAIL = N - NS * ROWS_A

_mesh = plsc.VectorSubcoreMesh(core_axis_name="c", subcore_axis_name="s")


HALF = NCHUNK // 2
NPAIR = HALF // 2


def _zero_rows(src_hbm, dst_sh, s):
    r0 = s * ROWS_A
    tail0 = NS * ROWS_A
    pltpu.sync_copy(src_hbm.at[pl.ds(r0, ROWS_A)], dst_sh.at[pl.ds(r0, ROWS_A)])

    @pl.when(s == NS - 1)
    def _():
        pltpu.sync_copy(src_hbm.at[pl.ds(tail0, ROWS_TAIL)],
                        dst_sh.at[pl.ds(tail0, ROWS_TAIL)])


def _writeback_rows(src_sh, dst_hbm, c, s):
    r0 = s * ROWS_A
    tail0 = NS * ROWS_A
    pltpu.sync_copy(src_sh.at[pl.ds(r0, ROWS_A)],
                    dst_hbm.at[c, pl.ds(r0, ROWS_A)])

    @pl.when(s == NS - 1)
    def _():
        pltpu.sync_copy(src_sh.at[pl.ds(tail0, ROWS_TAIL)],
                        dst_hbm.at[c, pl.ds(tail0, ROWS_TAIL)])


def _segsum_body(with_deg, *refs):
    if with_deg:
        (a_hbm, srcr, dstr, zrow, ones, out_hbm, deg_hbm,
         src_v, dst_v, rows0, rows1, agg_sh, sem0, sem1) = refs
    else:
        (a_hbm, srcr, dstr, zrow, out_hbm,
         src_v, dst_v, rows0, rows1, agg_sh, sem0, sem1) = refs

    c = lax.axis_index("c")
    s = lax.axis_index("s")
    wid = s * NC + c

    for half in range(2):
        h0 = half * HALF
        pltpu.sync_copy(srcr.at[wid, pl.ds(h0, HALF)], src_v)
        pltpu.sync_copy(dstr.at[wid, pl.ds(h0, HALF)], dst_v)
        pltpu.async_copy(a_hbm.at[src_v.at[0]], rows0, sem0)

        if half == 0:
            _zero_rows(zrow, agg_sh, s)
            plsc.subcore_barrier()

        def pair(p, carry):
            j = p * 2
            pltpu.async_copy(a_hbm.at[src_v.at[j + 1]], rows1, sem1)
            pltpu.make_async_copy(a_hbm.at[src_v.at[j]], rows0, sem0).wait()
            pltpu.sync_copy(rows0, agg_sh.at[dst_v.at[j]], add=True)

            @pl.when(p < NPAIR - 1)
            def _():
                pltpu.async_copy(a_hbm.at[src_v.at[j + 2]], rows0, sem0)

            pltpu.make_async_copy(a_hbm.at[src_v.at[j + 1]], rows1, sem1).wait()
            pltpu.sync_copy(rows1, agg_sh.at[dst_v.at[j + 1]], add=True)
            return carry
        lax.fori_loop(0, NPAIR, pair, 0)

    plsc.subcore_barrier()
    _writeback_rows(agg_sh, out_hbm, c, s)

    if with_deg:
        plsc.subcore_barrier()
        _zero_rows(zrow, agg_sh, s)
        pltpu.sync_copy(ones, rows0)
        plsc.subcore_barrier()

        for half in range(2):
            h0 = half * HALF
            pltpu.sync_copy(dstr.at[wid, pl.ds(h0, HALF)], dst_v)

            def dfire(j, carry):
                pltpu.async_copy(rows0, agg_sh.at[dst_v.at[j]], sem0, add=True)
                return carry
            lax.fori_loop(0, HALF, dfire, 0)

            def ddrain(j, carry):
                pltpu.make_async_copy(rows0, agg_sh.at[dst_v.at[j]],
                                      sem0).wait()
                return carry
            lax.fori_loop(0, HALF, ddrain, 0)

        plsc.subcore_barrier()
        _writeback_rows(agg_sh, deg_hbm, c, s)


_seg_scratch = [
    pltpu.VMEM((HALF, CHUNK), jnp.int32),
    pltpu.VMEM((HALF, CHUNK), jnp.int32),
    pltpu.VMEM((CHUNK, H), jnp.float32),
    pltpu.VMEM((CHUNK, H), jnp.float32),
    pltpu.VMEM_SHARED((N, H), jnp.float32),
    pltpu.SemaphoreType.DMA,
    pltpu.SemaphoreType.DMA,
]

_segsum_deg = pl.kernel(
    functools.partial(_segsum_body, True),
    out_type=(jax.ShapeDtypeStruct((NC, N, H), jnp.float32),
              jax.ShapeDtypeStruct((NC, N, H), jnp.float32)),
    mesh=_mesh,
    scratch_types=_seg_scratch,
)

_segsum = pl.kernel(
    functools.partial(_segsum_body, False),
    out_type=jax.ShapeDtypeStruct((NC, N, H), jnp.float32),
    mesh=_mesh,
    scratch_types=_seg_scratch,
)


BLK = 1000


def _dotT(x, w):
    return lax.dot_general(x, w, (((1,), (1,)), ((), ())),
                           preferred_element_type=jnp.float32)


def _combine2_body(sa_ref, sb_ref, da_ref, db_ref, x_ref, wl_ref, wr_ref,
                   bl_ref, h_ref):
    deg = jnp.maximum(da_ref[...][:, 0:1] + db_ref[...][:, 0:1], 1.0)
    mean = (sa_ref[...] + sb_ref[...]) / deg
    h_ref[...] = jnp.maximum(
        _dotT(mean, wl_ref[...]) + bl_ref[...] + _dotT(x_ref[...], wr_ref[...]),
        0.0)


def _combine2(sa, sb, da, db, x, wl, wr, bl):
    return pl.pallas_call(
        _combine2_body,
        grid=(N // BLK,),
        in_specs=[
            pl.BlockSpec((BLK, H), lambda i: (i, 0)),
            pl.BlockSpec((BLK, H), lambda i: (i, 0)),
            pl.BlockSpec((BLK, H), lambda i: (i, 0)),
            pl.BlockSpec((BLK, H), lambda i: (i, 0)),
            pl.BlockSpec((BLK, H), lambda i: (i, 0)),
            pl.BlockSpec((H, H), lambda i: (0, 0)),
            pl.BlockSpec((H, H), lambda i: (0, 0)),
            pl.BlockSpec((1, H), lambda i: (0, 0)),
        ],
        out_specs=pl.BlockSpec((BLK, H), lambda i: (i, 0)),
        out_shape=jax.ShapeDtypeStruct((N, H), jnp.float32),
    )(sa, sb, da, db, x, wl, wr, bl.reshape(1, H))


def _final_body(sa_ref, sb_ref, da_ref, db_ref, h_ref, wl_ref, wr_ref,
                bl_ref, wc_ref, bc_ref, o_ref):
    deg = jnp.maximum(da_ref[...][:, 0:1] + db_ref[...][:, 0:1], 1.0)
    mean = (sa_ref[...] + sb_ref[...]) / deg
    z = _dotT(mean, wl_ref[...]) + bl_ref[...] + _dotT(h_ref[...], wr_ref[...])
    o_ref[...] = _dotT(z, wc_ref[...]) + bc_ref[...]


def _final(sa, sb, da, db, h, wl, wr, bl, wc, bc):
    return pl.pallas_call(
        _final_body,
        grid=(N // BLK,),
        in_specs=[
            pl.BlockSpec((BLK, H), lambda i: (i, 0)),
            pl.BlockSpec((BLK, H), lambda i: (i, 0)),
            pl.BlockSpec((BLK, H), lambda i: (i, 0)),
            pl.BlockSpec((BLK, H), lambda i: (i, 0)),
            pl.BlockSpec((BLK, H), lambda i: (i, 0)),
            pl.BlockSpec((H, H), lambda i: (0, 0)),
            pl.BlockSpec((H, H), lambda i: (0, 0)),
            pl.BlockSpec((1, H), lambda i: (0, 0)),
            pl.BlockSpec((OUT, H), lambda i: (0, 0)),
            pl.BlockSpec((1, OUT), lambda i: (0, 0)),
        ],
        out_specs=pl.BlockSpec((BLK, OUT), lambda i: (i, 0)),
        out_shape=jax.ShapeDtypeStruct((N, OUT), jnp.float32),
    )(sa, sb, da, db, h, wl, wr, bl.reshape(1, H), wc, bc.reshape(1, OUT))


def kernel(x, edge_index, W1l, b1, W1r, W2l, b2, W2r, Wc, bc):
    srcr = edge_index[0].reshape(NW, NCHUNK, CHUNK)
    dstr = edge_index[1].reshape(NW, NCHUNK, CHUNK)
    zrow = jnp.zeros((N, H), jnp.float32)
    ones = jnp.ones((CHUNK, H), jnp.float32)

    s1p, degp = _segsum_deg(x, srcr, dstr, zrow, ones)
    h = _combine2(s1p[0], s1p[1], degp[0], degp[1], x, W1l, W1r, b1)
    s2p = _segsum(h, srcr, dstr, zrow)
    return _final(s2p[0], s2p[1], degp[0], degp[1], h, W2l, W2r, b2, Wc, bc)
